# baseline (speedup 1.0000x reference)
import jax
import jax.numpy as jnp
from jax import lax
from jax.experimental import pallas as pl
from jax.experimental.pallas import tpu as pltpu
from jax.experimental.pallas import tpu_sc as plsc

N_NODES = 10000
N_EDGES = 320000
D = 128
T = 7
NSEG = N_NODES * T
NSEG_PAD = 70400
NC = 2
NS = 16
NW = NC * NS
CH = 80
MW = 3 * CH
NBLK = N_EDGES // CH
EB = NBLK // NW
CB = NBLK // NS
CSUP = CB // 5
WSL = NSEG_PAD // NS
ROWS_A = 624
ROWS_EXTRA = N_NODES - NS * ROWS_A


def _mm_body(x_ref, w_ref, y_ref):
    y_ref[...] = jnp.dot(x_ref[...], w_ref[...],
                         preferred_element_type=jnp.float32)


def _compute_y(x, weights):
    wcat = weights.reshape(T, D, D).transpose(1, 0, 2).reshape(D, T * D)
    nb = 10
    bn = N_NODES // nb
    y2 = pl.pallas_call(
        _mm_body,
        grid=(nb,),
        in_specs=[
            pl.BlockSpec((bn, D), lambda b: (b, 0)),
            pl.BlockSpec((D, T * D), lambda b: (0, 0)),
        ],
        out_specs=pl.BlockSpec((bn, T * D), lambda b: (b, 0)),
        out_shape=jax.ShapeDtypeStruct((N_NODES, T * D), jnp.float32),
    )(x, wcat)
    return y2.reshape(NSEG, D)


def _add_body(a_ref, b_ref, o_ref):
    o_ref[...] = a_ref[...] + b_ref[...]


def _combine(partials):
    nb = 10
    bn = N_NODES // nb
    return pl.pallas_call(
        _add_body,
        grid=(nb,),
        in_specs=[pl.BlockSpec((bn, D), lambda i: (i, 0))] * 2,
        out_specs=pl.BlockSpec((bn, D), lambda i: (i, 0)),
        out_shape=jax.ShapeDtypeStruct((N_NODES, D), jnp.float32),
    )(partials[0], partials[1])


def _sc_body(y_hbm, meta_hbm, out_hbm,
             cnt_sh, out_sh,
             zbuf, onesb, cmeta, csegb,
             mbuf0, mbuf1, mbuf2, gidx0, gidx1, gidx2,
             rowb0, rowb1, rowb2, segb0, segb1, segb2,
             wvb0, wvb1, wvb2, rows0, rows1, rows2,
             sem_m0, sem_m1, sem_m2, sem_y0, sem_y1, sem_y2,
             sem_w0, sem_w1, sem_w2, sem_s0, sem_s1, sem_s2, sem_c):
    c = lax.axis_index("c")
    s = lax.axis_index("s")
    wid = s * NC + c

    zeros16 = jnp.zeros((16,), jnp.float32)
    ones16 = jnp.ones((16,), jnp.float32)

    s0 = (mbuf0, gidx0, rowb0, segb0, wvb0, rows0,
          sem_m0, sem_y0, sem_w0, sem_s0)
    s1 = (mbuf1, gidx1, rowb1, segb1, wvb1, rows1,
          sem_m1, sem_y1, sem_w1, sem_s1)
    s2 = (mbuf2, gidx2, rowb2, segb2, wvb2, rows2,
          sem_m2, sem_y2, sem_w2, sem_s2)

    def _z_w(i, _):
        zbuf[pl.ds(i * 16, 16)] = zeros16
        return 0
    lax.fori_loop(0, WSL // 16, _z_w, 0)
    pltpu.sync_copy(zbuf, cnt_sh.at[pl.ds(s * WSL, WSL)])

    def _z_rows(i, _):
        for j in range(8):
            rows0[i, pl.ds(j * 16, 16)] = zeros16
        return 0
    lax.fori_loop(0, CH, _z_rows, 0)
    r0 = s * ROWS_A
    for piece in range(7):
        pltpu.sync_copy(rows0.at[pl.ds(0, CH)],
                        out_sh.at[pl.ds(r0 + piece * CH, CH)])
    pltpu.sync_copy(rows0.at[pl.ds(0, ROWS_A - 7 * CH)],
                    out_sh.at[pl.ds(r0 + 7 * CH, ROWS_A - 7 * CH)])

    @pl.when(s == NS - 1)
    def _zero_extra():
        pltpu.sync_copy(rows0.at[pl.ds(0, ROWS_EXTRA)],
                        out_sh.at[pl.ds(NS * ROWS_A, ROWS_EXTRA)])

    for j in range(CH // 16):
        onesb[pl.ds(j * 16, 16)] = ones16

    plsc.subcore_barrier()

    cb0 = s * CB

    def _count_super(k, _):
        mo = (cb0 + k * 5) * MW
        pltpu.sync_copy(meta_hbm.at[pl.ds(mo, 5 * MW)], cmeta)
        for r in range(5):
            for g in range(CH // 16):
                rowv = cmeta[pl.ds(r * MW + g * 16, 16)]
                etv = cmeta[pl.ds(r * MW + 2 * CH + g * 16, 16)]
                csegb[r, pl.ds(g * 16, 16)] = rowv * T + etv
        for r in range(5):
            pltpu.async_copy(onesb, cnt_sh.at[csegb.at[r]], sem_c, add=True)
        for r in range(5):
            pltpu.make_async_copy(onesb, cnt_sh.at[csegb.at[r]], sem_c).wait()
        return 0
    lax.fori_loop(0, CSUP, _count_super, 0)

    plsc.subcore_barrier()

    pltpu.sync_copy(cnt_sh.at[pl.ds(s * WSL, WSL)], zbuf)

    def _w_body(i, _):
        sl = pl.ds(i * 16, 16)
        zbuf[sl] = 1.0 / jnp.maximum(zbuf[sl], 1.0)
        return 0
    lax.fori_loop(0, WSL // 16, _w_body, 0)
    pltpu.sync_copy(zbuf, cnt_sh.at[pl.ds(s * WSL, WSL)])

    plsc.subcore_barrier()

    blk0 = wid * EB

    def _meta_issue(j, P):
        pltpu.async_copy(meta_hbm.at[pl.ds((blk0 + j) * MW, MW)], P[0], P[6])

    def _meta_wait(P):
        pltpu.make_async_copy(meta_hbm.at[pl.ds(0, MW)], P[0], P[6]).wait()

    def _compute_issue(P):
        mbuf, gidx, rowb, segb, wvb, rows, _, sem_y, sem_w, _ = P
        for g in range(CH // 16):
            sl = pl.ds(g * 16, 16)
            rowv = mbuf[pl.ds(g * 16, 16)]
            colv = mbuf[pl.ds(CH + g * 16, 16)]
            etv = mbuf[pl.ds(2 * CH + g * 16, 16)]
            gidx[sl] = colv * T + etv
            segb[sl] = rowv * T + etv
            rowb[sl] = rowv
        pltpu.async_copy(cnt_sh.at[segb], wvb, sem_w)
        pltpu.async_copy(y_hbm.at[gidx], rows, sem_y)

    def _scale(P):
        wvb, rows = P[4], P[5]

        def _sc(g, _2):
            wv16 = wvb[pl.ds(g * 16, 16)]
            for l in range(16):
                wsc = wv16[l]
                e = g * 16 + l
                for j in range(8):
                    sl = pl.ds(j * 16, 16)
                    rows[e, sl] = rows[e, sl] * wsc
            return 0
        lax.fori_loop(0, CH // 16, _sc, 0)

    def _scatter_issue(P):
        pltpu.async_copy(P[5], out_sh.at[P[2]], P[9], add=True)

    def _scatter_wait(P):
        pltpu.make_async_copy(P[5], out_sh.at[P[2]], P[9]).wait()

    def _gathers_wait(P):
        _, gidx, _, segb, wvb, rows, _, sem_y, sem_w, _ = P
        pltpu.make_async_copy(y_hbm.at[gidx], rows, sem_y).wait()
        pltpu.make_async_copy(cnt_sh.at[segb], wvb, sem_w).wait()

    def _estep(j, P, Q, R, swait, donext, dometa2):
        if swait:
            _scatter_wait(Q)
        if donext:
            _meta_wait(Q)
            _compute_issue(Q)
        if dometa2:
            _meta_issue(j + 2, R)
        _gathers_wait(P)
        _scale(P)
        _scatter_issue(P)

    _meta_issue(0, s0)
    _meta_wait(s0)
    _compute_issue(s0)
    _meta_issue(1, s1)
    _estep(0, s0, s1, s2, swait=False, donext=True, dometa2=True)
    _estep(1, s1, s2, s0, swait=False, donext=True, dometa2=True)
    _estep(2, s2, s0, s1, swait=True, donext=True, dometa2=True)

    def _triple(q, _):
        j = 3 * q
        _estep(j, s0, s1, s2, swait=True, donext=True, dometa2=True)
        _estep(j + 1, s1, s2, s0, swait=True, donext=True, dometa2=True)
        _estep(j + 2, s2, s0, s1, swait=True, donext=True, dometa2=True)
        return 0
    lax.fori_loop(1, (EB - 5) // 3 + 1, _triple, 0)
    _estep(EB - 2, s0, s1, s2, swait=True, donext=True, dometa2=False)
    _estep(EB - 1, s1, s2, s0, swait=True, donext=False, dometa2=False)
    _scatter_wait(s0)
    _scatter_wait(s1)

    plsc.subcore_barrier()

    pltpu.sync_copy(out_sh.at[pl.ds(r0, ROWS_A)],
                    out_hbm.at[c, pl.ds(r0, ROWS_A)])

    @pl.when(s == NS - 1)
    def _write_extra():
        pltpu.sync_copy(out_sh.at[pl.ds(NS * ROWS_A, ROWS_EXTRA)],
                        out_hbm.at[c, pl.ds(NS * ROWS_A, ROWS_EXTRA)])


def _sc_edge_kernel(y, meta):
    mesh = plsc.VectorSubcoreMesh(core_axis_name="c", subcore_axis_name="s",
                                  num_cores=NC, num_subcores=NS)
    f = pl.kernel(
        _sc_body,
        out_type=jax.ShapeDtypeStruct((NC, N_NODES, D), jnp.float32),
        mesh=mesh,
        scratch_types=[
            pltpu.VMEM_SHARED((NSEG_PAD,), jnp.float32),
            pltpu.VMEM_SHARED((N_NODES, D), jnp.float32),
            pltpu.VMEM((WSL,), jnp.float32),
            pltpu.VMEM((CH,), jnp.float32),
            pltpu.VMEM((5 * MW,), jnp.int32),
            pltpu.VMEM((5, CH), jnp.int32),
            pltpu.VMEM((MW,), jnp.int32),
            pltpu.VMEM((MW,), jnp.int32),
            pltpu.VMEM((MW,), jnp.int32),
            pltpu.VMEM((CH,), jnp.int32),
            pltpu.VMEM((CH,), jnp.int32),
            pltpu.VMEM((CH,), jnp.int32),
            pltpu.VMEM((CH,), jnp.int32),
            pltpu.VMEM((CH,), jnp.int32),
            pltpu.VMEM((CH,), jnp.int32),
            pltpu.VMEM((CH,), jnp.int32),
            pltpu.VMEM((CH,), jnp.int32),
            pltpu.VMEM((CH,), jnp.int32),
            pltpu.VMEM((CH,), jnp.float32),
            pltpu.VMEM((CH,), jnp.float32),
            pltpu.VMEM((CH,), jnp.float32),
            pltpu.VMEM((CH, D), jnp.float32),
            pltpu.VMEM((CH, D), jnp.float32),
            pltpu.VMEM((CH, D), jnp.float32),
            pltpu.SemaphoreType.DMA,
            pltpu.SemaphoreType.DMA,
            pltpu.SemaphoreType.DMA,
            pltpu.SemaphoreType.DMA,
            pltpu.SemaphoreType.DMA,
            pltpu.SemaphoreType.DMA,
            pltpu.SemaphoreType.DMA,
            pltpu.SemaphoreType.DMA,
            pltpu.SemaphoreType.DMA,
            pltpu.SemaphoreType.DMA,
            pltpu.SemaphoreType.DMA,
            pltpu.SemaphoreType.DMA,
            pltpu.SemaphoreType.DMA,
        ],
        compiler_params=pltpu.CompilerParams(needs_layout_passes=False),
    )
    return f(y, meta)


def kernel(x, edge_index, edge_type, weights):
    row = edge_index[0].astype(jnp.int32)
    col = edge_index[1].astype(jnp.int32)
    et = edge_type.astype(jnp.int32)
    meta = jnp.stack([row.reshape(NBLK, CH), col.reshape(NBLK, CH),
                      et.reshape(NBLK, CH)], axis=1).reshape(-1)
    y = _compute_y(x, weights)
    partials = _sc_edge_kernel(y, meta)
    return _combine(partials)


# speedup vs baseline: 1.0246x; 1.0246x over previous
import jax
import jax.numpy as jnp
from jax import lax
from jax.experimental import pallas as pl
from jax.experimental.pallas import tpu as pltpu
from jax.experimental.pallas import tpu_sc as plsc

N_NODES = 10000
N_EDGES = 320000
D = 128
T = 7
NSEG = N_NODES * T
NSEG_PAD = 70400
NC = 2
NS = 16
NW = NC * NS
CH = 80
NBLK = N_EDGES // CH
EB = NBLK // NW
CB = NBLK // NS
CSUP = CB // 5
WSL = NSEG_PAD // NS
ROWS_A = 624
ROWS_EXTRA = N_NODES - NS * ROWS_A


def _mm_body(x_ref, w_ref, y_ref):
    y_ref[...] = jnp.dot(x_ref[...], w_ref[...],
                         preferred_element_type=jnp.float32)


def _compute_y(x, weights):
    wcat = weights.reshape(T, D, D).transpose(1, 0, 2).reshape(D, T * D)
    nb = 10
    bn = N_NODES // nb
    y2 = pl.pallas_call(
        _mm_body,
        grid=(nb,),
        in_specs=[
            pl.BlockSpec((bn, D), lambda b: (b, 0)),
            pl.BlockSpec((D, T * D), lambda b: (0, 0)),
        ],
        out_specs=pl.BlockSpec((bn, T * D), lambda b: (b, 0)),
        out_shape=jax.ShapeDtypeStruct((N_NODES, T * D), jnp.float32),
    )(x, wcat)
    return y2.reshape(NSEG, D)


def _add3_body(a_ref, b_ref, o_ref):
    o_ref[...] = a_ref[0] + b_ref[0]


def _combine(partials):
    nb = 10
    bn = N_NODES // nb
    return pl.pallas_call(
        _add3_body,
        grid=(nb,),
        in_specs=[
            pl.BlockSpec((1, bn, D), lambda i: (0, i, 0)),
            pl.BlockSpec((1, bn, D), lambda i: (1, i, 0)),
        ],
        out_specs=pl.BlockSpec((bn, D), lambda i: (i, 0)),
        out_shape=jax.ShapeDtypeStruct((N_NODES, D), jnp.float32),
    )(partials, partials)


def _sc_body(y_hbm, gidx_hbm, seg_hbm, out_hbm,
             cnt_sh, out_sh,
             zbuf, onesb, cflat, csegb,
             gidx0, gidx1, gidx2, rowb0, rowb1, rowb2,
             segb0, segb1, segb2, wvb0, wvb1, wvb2,
             rows0, rows1, rows2,
             sem_m0, sem_m1, sem_m2, sem_y0, sem_y1, sem_y2,
             sem_w0, sem_w1, sem_w2, sem_s0, sem_s1, sem_s2, sem_c):
    c = lax.axis_index("c")
    s = lax.axis_index("s")
    wid = s * NC + c

    zeros16 = jnp.zeros((16,), jnp.float32)
    ones16 = jnp.ones((16,), jnp.float32)

    s0 = (gidx0, rowb0, segb0, wvb0, rows0, sem_m0, sem_y0, sem_w0, sem_s0)
    s1 = (gidx1, rowb1, segb1, wvb1, rows1, sem_m1, sem_y1, sem_w1, sem_s1)
    s2 = (gidx2, rowb2, segb2, wvb2, rows2, sem_m2, sem_y2, sem_w2, sem_s2)

    def _z_w(i, _):
        zbuf[pl.ds(i * 16, 16)] = zeros16
        return 0
    lax.fori_loop(0, WSL // 16, _z_w, 0)
    pltpu.sync_copy(zbuf, cnt_sh.at[pl.ds(s * WSL, WSL)])

    def _z_rows(i, _):
        for j in range(8):
            rows0[i, pl.ds(j * 16, 16)] = zeros16
        return 0
    lax.fori_loop(0, CH, _z_rows, 0)
    r0 = s * ROWS_A
    for piece in range(7):
        pltpu.sync_copy(rows0.at[pl.ds(0, CH)],
                        out_sh.at[pl.ds(r0 + piece * CH, CH)])
    pltpu.sync_copy(rows0.at[pl.ds(0, ROWS_A - 7 * CH)],
                    out_sh.at[pl.ds(r0 + 7 * CH, ROWS_A - 7 * CH)])

    @pl.when(s == NS - 1)
    def _zero_extra():
        pltpu.sync_copy(rows0.at[pl.ds(0, ROWS_EXTRA)],
                        out_sh.at[pl.ds(NS * ROWS_A, ROWS_EXTRA)])

    for j in range(CH // 16):
        onesb[pl.ds(j * 16, 16)] = ones16

    plsc.subcore_barrier()

    cb0 = s * CB

    def _count_super(k, _):
        eo = (cb0 + k * 5) * CH
        pltpu.sync_copy(seg_hbm.at[pl.ds(eo, 5 * CH)], cflat)
        for r in range(5):
            for g in range(CH // 16):
                csegb[r, pl.ds(g * 16, 16)] = cflat[pl.ds(r * CH + g * 16, 16)]
        for r in range(5):
            pltpu.async_copy(onesb, cnt_sh.at[csegb.at[r]], sem_c, add=True)
        for r in range(5):
            pltpu.make_async_copy(onesb, cnt_sh.at[csegb.at[r]], sem_c).wait()
        return 0
    lax.fori_loop(0, CSUP, _count_super, 0)

    plsc.subcore_barrier()

    pltpu.sync_copy(cnt_sh.at[pl.ds(s * WSL, WSL)], zbuf)

    def _w_body(i, _):
        sl = pl.ds(i * 16, 16)
        zbuf[sl] = 1.0 / jnp.maximum(zbuf[sl], 1.0)
        return 0
    lax.fori_loop(0, WSL // 16, _w_body, 0)
    pltpu.sync_copy(zbuf, cnt_sh.at[pl.ds(s * WSL, WSL)])

    plsc.subcore_barrier()

    blk0 = wid * EB

    def _meta_issue(j, P):
        eo = (blk0 + j) * CH
        pltpu.async_copy(gidx_hbm.at[pl.ds(eo, CH)], P[0], P[5])
        pltpu.async_copy(seg_hbm.at[pl.ds(eo, CH)], P[2], P[5])

    def _meta_wait(P):
        pltpu.make_async_copy(gidx_hbm.at[pl.ds(0, CH)], P[0], P[5]).wait()
        pltpu.make_async_copy(seg_hbm.at[pl.ds(0, CH)], P[2], P[5]).wait()

    def _compute_issue(P):
        gidx, rowb, segb, wvb, rows, _, sem_y, sem_w, _ = P
        for g in range(CH // 16):
            sl = pl.ds(g * 16, 16)
            rowb[sl] = segb[sl] // T
        pltpu.async_copy(cnt_sh.at[segb], wvb, sem_w)
        pltpu.async_copy(y_hbm.at[gidx], rows, sem_y)

    def _scale(P):
        wvb, rows = P[3], P[4]

        def _sc(g, _2):
            wv16 = wvb[pl.ds(g * 16, 16)]
            for l in range(16):
                wsc = wv16[l]
                e = g * 16 + l
                for j in range(8):
                    sl = pl.ds(j * 16, 16)
                    rows[e, sl] = rows[e, sl] * wsc
            return 0
        lax.fori_loop(0, CH // 16, _sc, 0)

    def _scatter_issue(P):
        pltpu.async_copy(P[4], out_sh.at[P[1]], P[8], add=True)

    def _scatter_wait(P):
        pltpu.make_async_copy(P[4], out_sh.at[P[1]], P[8]).wait()

    def _gathers_wait(P):
        gidx, _, segb, wvb, rows, _, sem_y, sem_w, _ = P
        pltpu.make_async_copy(y_hbm.at[gidx], rows, sem_y).wait()
        pltpu.make_async_copy(cnt_sh.at[segb], wvb, sem_w).wait()

    def _estep(j, P, Q, R, swait, donext, dometa2):
        if swait:
            _scatter_wait(Q)
        if donext:
            _meta_wait(Q)
            _compute_issue(Q)
        if dometa2:
            _meta_issue(j + 2, R)
        _gathers_wait(P)
        _scale(P)
        _scatter_issue(P)

    _meta_issue(0, s0)
    _meta_wait(s0)
    _compute_issue(s0)
    _meta_issue(1, s1)
    _estep(0, s0, s1, s2, swait=False, donext=True, dometa2=True)
    _estep(1, s1, s2, s0, swait=False, donext=True, dometa2=True)
    _estep(2, s2, s0, s1, swait=True, donext=True, dometa2=True)

    def _triple(q, _):
        j = 3 * q
        _estep(j, s0, s1, s2, swait=True, donext=True, dometa2=True)
        _estep(j + 1, s1, s2, s0, swait=True, donext=True, dometa2=True)
        _estep(j + 2, s2, s0, s1, swait=True, donext=True, dometa2=True)
        return 0
    lax.fori_loop(1, (EB - 5) // 3 + 1, _triple, 0)
    _estep(EB - 2, s0, s1, s2, swait=True, donext=True, dometa2=False)
    _estep(EB - 1, s1, s2, s0, swait=True, donext=False, dometa2=False)
    _scatter_wait(s0)
    _scatter_wait(s1)

    plsc.subcore_barrier()

    pltpu.sync_copy(out_sh.at[pl.ds(r0, ROWS_A)],
                    out_hbm.at[c, pl.ds(r0, ROWS_A)])

    @pl.when(s == NS - 1)
    def _write_extra():
        pltpu.sync_copy(out_sh.at[pl.ds(NS * ROWS_A, ROWS_EXTRA)],
                        out_hbm.at[c, pl.ds(NS * ROWS_A, ROWS_EXTRA)])


def _sc_edge_kernel(y, gidx, seg):
    mesh = plsc.VectorSubcoreMesh(core_axis_name="c", subcore_axis_name="s",
                                  num_cores=NC, num_subcores=NS)
    f = pl.kernel(
        _sc_body,
        out_type=jax.ShapeDtypeStruct((NC, N_NODES, D), jnp.float32),
        mesh=mesh,
        scratch_types=[
            pltpu.VMEM_SHARED((NSEG_PAD,), jnp.float32),
            pltpu.VMEM_SHARED((N_NODES, D), jnp.float32),
            pltpu.VMEM((WSL,), jnp.float32),
            pltpu.VMEM((CH,), jnp.float32),
            pltpu.VMEM((5 * CH,), jnp.int32),
            pltpu.VMEM((5, CH), jnp.int32),
            pltpu.VMEM((CH,), jnp.int32),
            pltpu.VMEM((CH,), jnp.int32),
            pltpu.VMEM((CH,), jnp.int32),
            pltpu.VMEM((CH,), jnp.int32),
            pltpu.VMEM((CH,), jnp.int32),
            pltpu.VMEM((CH,), jnp.int32),
            pltpu.VMEM((CH,), jnp.int32),
            pltpu.VMEM((CH,), jnp.int32),
            pltpu.VMEM((CH,), jnp.int32),
            pltpu.VMEM((CH,), jnp.float32),
            pltpu.VMEM((CH,), jnp.float32),
            pltpu.VMEM((CH,), jnp.float32),
            pltpu.VMEM((CH, D), jnp.float32),
            pltpu.VMEM((CH, D), jnp.float32),
            pltpu.VMEM((CH, D), jnp.float32),
            pltpu.SemaphoreType.DMA,
            pltpu.SemaphoreType.DMA,
            pltpu.SemaphoreType.DMA,
            pltpu.SemaphoreType.DMA,
            pltpu.SemaphoreType.DMA,
            pltpu.SemaphoreType.DMA,
            pltpu.SemaphoreType.DMA,
            pltpu.SemaphoreType.DMA,
            pltpu.SemaphoreType.DMA,
            pltpu.SemaphoreType.DMA,
            pltpu.SemaphoreType.DMA,
            pltpu.SemaphoreType.DMA,
            pltpu.SemaphoreType.DMA,
        ],
        compiler_params=pltpu.CompilerParams(needs_layout_passes=False),
    )
    return f(y, gidx, seg)


def kernel(x, edge_index, edge_type, weights):
    row = edge_index[0].astype(jnp.int32)
    col = edge_index[1].astype(jnp.int32)
    et = edge_type.astype(jnp.int32)
    gidx = col * T + et
    seg = row * T + et
    y = _compute_y(x, weights)
    partials = _sc_edge_kernel(y, gidx, seg)
    return _combine(partials)


# speedup vs baseline: 1.1903x; 1.1618x over previous
import jax
import jax.numpy as jnp
from jax import lax
from jax.experimental import pallas as pl
from jax.experimental.pallas import tpu as pltpu
from jax.experimental.pallas import tpu_sc as plsc

N_NODES = 10000
N_EDGES = 320000
D = 128
T = 7
NSEG = N_NODES * T
NSEG_PAD = 70400
NC = 2
NS = 16
NW = NC * NS
CH = 80
NBLK = N_EDGES // CH
EB = NBLK // NW
CB = NBLK // NS
CSUP = CB // 5
WSL = NSEG_PAD // NS
ROWS_A = 624
ROWS_EXTRA = N_NODES - NS * ROWS_A


def _mm_body(x_ref, w_ref, y_ref):
    y_ref[...] = jnp.dot(x_ref[...], w_ref[...],
                         preferred_element_type=jnp.float32)


def _compute_y(x, weights):
    wcat = weights.reshape(T, D, D).transpose(1, 0, 2).reshape(D, T * D)
    nb = 10
    bn = N_NODES // nb
    y2 = pl.pallas_call(
        _mm_body,
        grid=(nb,),
        in_specs=[
            pl.BlockSpec((bn, D), lambda b: (b, 0)),
            pl.BlockSpec((D, T * D), lambda b: (0, 0)),
        ],
        out_specs=pl.BlockSpec((bn, T * D), lambda b: (b, 0)),
        out_shape=jax.ShapeDtypeStruct((N_NODES, T * D), jnp.float32),
    )(x, wcat)
    return y2.reshape(NSEG, D)


def _add3_body(a_ref, b_ref, o_ref):
    o_ref[...] = a_ref[0] + b_ref[0]


def _combine(partials):
    nb = 10
    bn = N_NODES // nb
    return pl.pallas_call(
        _add3_body,
        grid=(nb,),
        in_specs=[
            pl.BlockSpec((1, bn, D), lambda i: (0, i, 0)),
            pl.BlockSpec((1, bn, D), lambda i: (1, i, 0)),
        ],
        out_specs=pl.BlockSpec((bn, D), lambda i: (i, 0)),
        out_shape=jax.ShapeDtypeStruct((N_NODES, D), jnp.float32),
    )(partials, partials)


def _sc_body(y_hbm, gidx_hbm, seg_hbm, row_hbm, out_hbm,
             cnt_sh, out_sh,
             zbuf, onesb, cflat, csegb,
             gidx0, gidx1, gidx2, rowb0, rowb1, rowb2,
             segb0, segb1, segb2, wvb0, wvb1, wvb2,
             rows0, rows1, rows2,
             sem_m0, sem_m1, sem_m2, sem_y0, sem_y1, sem_y2,
             sem_w0, sem_w1, sem_w2, sem_s0, sem_s1, sem_s2, sem_c):
    c = lax.axis_index("c")
    s = lax.axis_index("s")
    wid = s * NC + c

    zeros16 = jnp.zeros((16,), jnp.float32)
    ones16 = jnp.ones((16,), jnp.float32)

    s0 = (gidx0, rowb0, segb0, wvb0, rows0, sem_m0, sem_y0, sem_w0, sem_s0)
    s1 = (gidx1, rowb1, segb1, wvb1, rows1, sem_m1, sem_y1, sem_w1, sem_s1)
    s2 = (gidx2, rowb2, segb2, wvb2, rows2, sem_m2, sem_y2, sem_w2, sem_s2)

    def _z_w(i, _):
        zbuf[pl.ds(i * 16, 16)] = zeros16
        return 0
    lax.fori_loop(0, WSL // 16, _z_w, 0)
    pltpu.sync_copy(zbuf, cnt_sh.at[pl.ds(s * WSL, WSL)])

    def _z_rows(i, _):
        for j in range(8):
            rows0[i, pl.ds(j * 16, 16)] = zeros16
        return 0
    lax.fori_loop(0, CH, _z_rows, 0)
    r0 = s * ROWS_A
    for piece in range(7):
        pltpu.sync_copy(rows0.at[pl.ds(0, CH)],
                        out_sh.at[pl.ds(r0 + piece * CH, CH)])
    pltpu.sync_copy(rows0.at[pl.ds(0, ROWS_A - 7 * CH)],
                    out_sh.at[pl.ds(r0 + 7 * CH, ROWS_A - 7 * CH)])

    @pl.when(s == NS - 1)
    def _zero_extra():
        pltpu.sync_copy(rows0.at[pl.ds(0, ROWS_EXTRA)],
                        out_sh.at[pl.ds(NS * ROWS_A, ROWS_EXTRA)])

    for j in range(CH // 16):
        onesb[pl.ds(j * 16, 16)] = ones16

    plsc.subcore_barrier()

    cb0 = s * CB

    def _count_super(k, _):
        eo = (cb0 + k * 5) * CH
        pltpu.sync_copy(seg_hbm.at[pl.ds(eo, 5 * CH)], cflat)
        for r in range(5):
            for g in range(CH // 16):
                csegb[r, pl.ds(g * 16, 16)] = cflat[pl.ds(r * CH + g * 16, 16)]
        for r in range(5):
            pltpu.async_copy(onesb, cnt_sh.at[csegb.at[r]], sem_c, add=True)
        for r in range(5):
            pltpu.make_async_copy(onesb, cnt_sh.at[csegb.at[r]], sem_c).wait()
        return 0
    lax.fori_loop(0, CSUP, _count_super, 0)

    plsc.subcore_barrier()

    pltpu.sync_copy(cnt_sh.at[pl.ds(s * WSL, WSL)], zbuf)

    def _w_body(i, _):
        sl = pl.ds(i * 16, 16)
        zbuf[sl] = 1.0 / jnp.maximum(zbuf[sl], 1.0)
        return 0
    lax.fori_loop(0, WSL // 16, _w_body, 0)
    pltpu.sync_copy(zbuf, cnt_sh.at[pl.ds(s * WSL, WSL)])

    plsc.subcore_barrier()

    blk0 = wid * EB

    def _meta_issue(j, P):
        eo = (blk0 + j) * CH
        pltpu.async_copy(gidx_hbm.at[pl.ds(eo, CH)], P[0], P[5])
        pltpu.async_copy(seg_hbm.at[pl.ds(eo, CH)], P[2], P[5])
        pltpu.async_copy(row_hbm.at[pl.ds(eo, CH)], P[1], P[5])

    def _meta_wait(P):
        pltpu.make_async_copy(gidx_hbm.at[pl.ds(0, CH)], P[0], P[5]).wait()
        pltpu.make_async_copy(seg_hbm.at[pl.ds(0, CH)], P[2], P[5]).wait()
        pltpu.make_async_copy(row_hbm.at[pl.ds(0, CH)], P[1], P[5]).wait()

    def _compute_issue(P):
        gidx, rowb, segb, wvb, rows, _, sem_y, sem_w, _ = P
        pltpu.async_copy(cnt_sh.at[segb], wvb, sem_w)
        pltpu.async_copy(y_hbm.at[gidx], rows, sem_y)

    def _scale(P):
        wvb, rows = P[3], P[4]

        def _sc(g, _2):
            wv16 = wvb[pl.ds(g * 16, 16)]
            for l in range(16):
                wsc = wv16[l]
                e = g * 16 + l
                for j in range(8):
                    sl = pl.ds(j * 16, 16)
                    rows[e, sl] = rows[e, sl] * wsc
            return 0
        lax.fori_loop(0, CH // 16, _sc, 0)

    def _scatter_issue(P):
        pltpu.async_copy(P[4], out_sh.at[P[1]], P[8], add=True)

    def _scatter_wait(P):
        pltpu.make_async_copy(P[4], out_sh.at[P[1]], P[8]).wait()

    def _gathers_wait(P):
        gidx, _, segb, wvb, rows, _, sem_y, sem_w, _ = P
        pltpu.make_async_copy(y_hbm.at[gidx], rows, sem_y).wait()
        pltpu.make_async_copy(cnt_sh.at[segb], wvb, sem_w).wait()

    def _estep(j, P, Q, R, swait, donext, dometa2):
        if swait:
            _scatter_wait(Q)
        if donext:
            _meta_wait(Q)
            _compute_issue(Q)
        if dometa2:
            _meta_issue(j + 2, R)
        _gathers_wait(P)
        _scale(P)
        _scatter_issue(P)

    _meta_issue(0, s0)
    _meta_wait(s0)
    _compute_issue(s0)
    _meta_issue(1, s1)
    _estep(0, s0, s1, s2, swait=False, donext=True, dometa2=True)
    _estep(1, s1, s2, s0, swait=False, donext=True, dometa2=True)
    _estep(2, s2, s0, s1, swait=True, donext=True, dometa2=True)

    def _triple(q, _):
        j = 3 * q
        _estep(j, s0, s1, s2, swait=True, donext=True, dometa2=True)
        _estep(j + 1, s1, s2, s0, swait=True, donext=True, dometa2=True)
        _estep(j + 2, s2, s0, s1, swait=True, donext=True, dometa2=True)
        return 0
    lax.fori_loop(1, (EB - 5) // 3 + 1, _triple, 0)
    _estep(EB - 2, s0, s1, s2, swait=True, donext=True, dometa2=False)
    _estep(EB - 1, s1, s2, s0, swait=True, donext=False, dometa2=False)
    _scatter_wait(s0)
    _scatter_wait(s1)

    plsc.subcore_barrier()

    pltpu.sync_copy(out_sh.at[pl.ds(r0, ROWS_A)],
                    out_hbm.at[c, pl.ds(r0, ROWS_A)])

    @pl.when(s == NS - 1)
    def _write_extra():
        pltpu.sync_copy(out_sh.at[pl.ds(NS * ROWS_A, ROWS_EXTRA)],
                        out_hbm.at[c, pl.ds(NS * ROWS_A, ROWS_EXTRA)])


def _sc_edge_kernel(y, gidx, seg, row):
    mesh = plsc.VectorSubcoreMesh(core_axis_name="c", subcore_axis_name="s",
                                  num_cores=NC, num_subcores=NS)
    f = pl.kernel(
        _sc_body,
        out_type=jax.ShapeDtypeStruct((NC, N_NODES, D), jnp.float32),
        mesh=mesh,
        scratch_types=[
            pltpu.VMEM_SHARED((NSEG_PAD,), jnp.float32),
            pltpu.VMEM_SHARED((N_NODES, D), jnp.float32),
            pltpu.VMEM((WSL,), jnp.float32),
            pltpu.VMEM((CH,), jnp.float32),
            pltpu.VMEM((5 * CH,), jnp.int32),
            pltpu.VMEM((5, CH), jnp.int32),
            pltpu.VMEM((CH,), jnp.int32),
            pltpu.VMEM((CH,), jnp.int32),
            pltpu.VMEM((CH,), jnp.int32),
            pltpu.VMEM((CH,), jnp.int32),
            pltpu.VMEM((CH,), jnp.int32),
            pltpu.VMEM((CH,), jnp.int32),
            pltpu.VMEM((CH,), jnp.int32),
            pltpu.VMEM((CH,), jnp.int32),
            pltpu.VMEM((CH,), jnp.int32),
            pltpu.VMEM((CH,), jnp.float32),
            pltpu.VMEM((CH,), jnp.float32),
            pltpu.VMEM((CH,), jnp.float32),
            pltpu.VMEM((CH, D), jnp.float32),
            pltpu.VMEM((CH, D), jnp.float32),
            pltpu.VMEM((CH, D), jnp.float32),
            pltpu.SemaphoreType.DMA,
            pltpu.SemaphoreType.DMA,
            pltpu.SemaphoreType.DMA,
            pltpu.SemaphoreType.DMA,
            pltpu.SemaphoreType.DMA,
            pltpu.SemaphoreType.DMA,
            pltpu.SemaphoreType.DMA,
            pltpu.SemaphoreType.DMA,
            pltpu.SemaphoreType.DMA,
            pltpu.SemaphoreType.DMA,
            pltpu.SemaphoreType.DMA,
            pltpu.SemaphoreType.DMA,
            pltpu.SemaphoreType.DMA,
        ],
        compiler_params=pltpu.CompilerParams(needs_layout_passes=False),
    )
    return f(y, gidx, seg, row)


def kernel(x, edge_index, edge_type, weights):
    row = edge_index[0].astype(jnp.int32)
    col = edge_index[1].astype(jnp.int32)
    et = edge_type.astype(jnp.int32)
    gidx = col * T + et
    seg = row * T + et
    y = _compute_y(x, weights)
    partials = _sc_edge_kernel(y, gidx, seg, row)
    return _combine(partials)


# speedup vs baseline: 1.2065x; 1.0136x over previous
import jax
import jax.numpy as jnp
from jax import lax
from jax.experimental import pallas as pl
from jax.experimental.pallas import tpu as pltpu
from jax.experimental.pallas import tpu_sc as plsc

N_NODES = 10000
N_EDGES = 320000
D = 128
T = 7
NSEG = N_NODES * T
NSEG_PAD = 70400
NC = 2
NS = 16
NW = NC * NS
CH = 80
NBLK = N_EDGES // CH
EB = NBLK // NW
CB = NBLK // NS
CSUP = CB // 5
WSL = NSEG_PAD // NS
ROWS_A = 624
ROWS_EXTRA = N_NODES - NS * ROWS_A


def _mm_body(x_ref, w_ref, y_ref):
    y_ref[...] = jnp.dot(x_ref[...], w_ref[...],
                         preferred_element_type=jnp.float32)


def _compute_y(x, weights):
    nb = 10
    bn = N_NODES // nb
    return pl.pallas_call(
        _mm_body,
        grid=(nb, T),
        in_specs=[
            pl.BlockSpec((bn, D), lambda b, t: (b, 0)),
            pl.BlockSpec((D, D), lambda b, t: (t, 0)),
        ],
        out_specs=pl.BlockSpec((bn, D), lambda b, t: (t * nb + b, 0)),
        out_shape=jax.ShapeDtypeStruct((NSEG, D), jnp.float32),
    )(x, weights)


def _add3_body(a_ref, b_ref, o_ref):
    o_ref[...] = a_ref[0] + b_ref[0]


def _combine(partials):
    nb = 10
    bn = N_NODES // nb
    return pl.pallas_call(
        _add3_body,
        grid=(nb,),
        in_specs=[
            pl.BlockSpec((1, bn, D), lambda i: (0, i, 0)),
            pl.BlockSpec((1, bn, D), lambda i: (1, i, 0)),
        ],
        out_specs=pl.BlockSpec((bn, D), lambda i: (i, 0)),
        out_shape=jax.ShapeDtypeStruct((N_NODES, D), jnp.float32),
    )(partials, partials)


def _sc_body(y_hbm, gidx_hbm, seg_hbm, row_hbm, out_hbm,
             cnt_sh, out_sh,
             zbuf, onesb, cflat, csegb,
             gidx0, gidx1, gidx2, rowb0, rowb1, rowb2,
             segb0, segb1, segb2, wvb0, wvb1, wvb2,
             rows0, rows1, rows2,
             sem_m0, sem_m1, sem_m2, sem_y0, sem_y1, sem_y2,
             sem_w0, sem_w1, sem_w2, sem_s0, sem_s1, sem_s2, sem_c):
    c = lax.axis_index("c")
    s = lax.axis_index("s")
    wid = s * NC + c

    zeros16 = jnp.zeros((16,), jnp.float32)
    ones16 = jnp.ones((16,), jnp.float32)

    s0 = (gidx0, rowb0, segb0, wvb0, rows0, sem_m0, sem_y0, sem_w0, sem_s0)
    s1 = (gidx1, rowb1, segb1, wvb1, rows1, sem_m1, sem_y1, sem_w1, sem_s1)
    s2 = (gidx2, rowb2, segb2, wvb2, rows2, sem_m2, sem_y2, sem_w2, sem_s2)

    def _z_w(i, _):
        zbuf[pl.ds(i * 16, 16)] = zeros16
        return 0
    lax.fori_loop(0, WSL // 16, _z_w, 0)
    pltpu.sync_copy(zbuf, cnt_sh.at[pl.ds(s * WSL, WSL)])

    def _z_rows(i, _):
        for j in range(8):
            rows0[i, pl.ds(j * 16, 16)] = zeros16
        return 0
    lax.fori_loop(0, CH, _z_rows, 0)
    r0 = s * ROWS_A
    for piece in range(7):
        pltpu.sync_copy(rows0.at[pl.ds(0, CH)],
                        out_sh.at[pl.ds(r0 + piece * CH, CH)])
    pltpu.sync_copy(rows0.at[pl.ds(0, ROWS_A - 7 * CH)],
                    out_sh.at[pl.ds(r0 + 7 * CH, ROWS_A - 7 * CH)])

    @pl.when(s == NS - 1)
    def _zero_extra():
        pltpu.sync_copy(rows0.at[pl.ds(0, ROWS_EXTRA)],
                        out_sh.at[pl.ds(NS * ROWS_A, ROWS_EXTRA)])

    for j in range(CH // 16):
        onesb[pl.ds(j * 16, 16)] = ones16

    plsc.subcore_barrier()

    cb0 = s * CB

    def _count_super(k, _):
        eo = (cb0 + k * 5) * CH
        pltpu.sync_copy(seg_hbm.at[pl.ds(eo, 5 * CH)], cflat)
        for r in range(5):
            for g in range(CH // 16):
                csegb[r, pl.ds(g * 16, 16)] = cflat[pl.ds(r * CH + g * 16, 16)]
        for r in range(5):
            pltpu.async_copy(onesb, cnt_sh.at[csegb.at[r]], sem_c, add=True)
        for r in range(5):
            pltpu.make_async_copy(onesb, cnt_sh.at[csegb.at[r]], sem_c).wait()
        return 0
    lax.fori_loop(0, CSUP, _count_super, 0)

    plsc.subcore_barrier()

    pltpu.sync_copy(cnt_sh.at[pl.ds(s * WSL, WSL)], zbuf)

    def _w_body(i, _):
        sl = pl.ds(i * 16, 16)
        zbuf[sl] = 1.0 / jnp.maximum(zbuf[sl], 1.0)
        return 0
    lax.fori_loop(0, WSL // 16, _w_body, 0)
    pltpu.sync_copy(zbuf, cnt_sh.at[pl.ds(s * WSL, WSL)])

    plsc.subcore_barrier()

    blk0 = wid * EB

    def _meta_issue(j, P):
        eo = (blk0 + j) * CH
        pltpu.async_copy(gidx_hbm.at[pl.ds(eo, CH)], P[0], P[5])
        pltpu.async_copy(seg_hbm.at[pl.ds(eo, CH)], P[2], P[5])
        pltpu.async_copy(row_hbm.at[pl.ds(eo, CH)], P[1], P[5])

    def _meta_wait(P):
        pltpu.make_async_copy(gidx_hbm.at[pl.ds(0, CH)], P[0], P[5]).wait()
        pltpu.make_async_copy(seg_hbm.at[pl.ds(0, CH)], P[2], P[5]).wait()
        pltpu.make_async_copy(row_hbm.at[pl.ds(0, CH)], P[1], P[5]).wait()

    def _compute_issue(P):
        gidx, rowb, segb, wvb, rows, _, sem_y, sem_w, _ = P
        pltpu.async_copy(cnt_sh.at[segb], wvb, sem_w)
        pltpu.async_copy(y_hbm.at[gidx], rows, sem_y)

    def _scale(P):
        wvb, rows = P[3], P[4]

        def _sc(g, _2):
            wv16 = wvb[pl.ds(g * 16, 16)]
            for l in range(16):
                wsc = wv16[l]
                e = g * 16 + l
                for j in range(8):
                    sl = pl.ds(j * 16, 16)
                    rows[e, sl] = rows[e, sl] * wsc
            return 0
        lax.fori_loop(0, CH // 16, _sc, 0)

    def _scatter_issue(P):
        pltpu.async_copy(P[4], out_sh.at[P[1]], P[8], add=True)

    def _scatter_wait(P):
        pltpu.make_async_copy(P[4], out_sh.at[P[1]], P[8]).wait()

    def _gathers_wait(P):
        gidx, _, segb, wvb, rows, _, sem_y, sem_w, _ = P
        pltpu.make_async_copy(y_hbm.at[gidx], rows, sem_y).wait()
        pltpu.make_async_copy(cnt_sh.at[segb], wvb, sem_w).wait()

    def _estep(j, P, Q, R, swait, donext, dometa2):
        if swait:
            _scatter_wait(Q)
        if donext:
            _meta_wait(Q)
            _compute_issue(Q)
        if dometa2:
            _meta_issue(j + 2, R)
        _gathers_wait(P)
        _scale(P)
        _scatter_issue(P)

    _meta_issue(0, s0)
    _meta_wait(s0)
    _compute_issue(s0)
    _meta_issue(1, s1)
    _estep(0, s0, s1, s2, swait=False, donext=True, dometa2=True)
    _estep(1, s1, s2, s0, swait=False, donext=True, dometa2=True)
    _estep(2, s2, s0, s1, swait=True, donext=True, dometa2=True)

    def _triple(q, _):
        j = 3 * q
        _estep(j, s0, s1, s2, swait=True, donext=True, dometa2=True)
        _estep(j + 1, s1, s2, s0, swait=True, donext=True, dometa2=True)
        _estep(j + 2, s2, s0, s1, swait=True, donext=True, dometa2=True)
        return 0
    lax.fori_loop(1, (EB - 5) // 3 + 1, _triple, 0)
    _estep(EB - 2, s0, s1, s2, swait=True, donext=True, dometa2=False)
    _estep(EB - 1, s1, s2, s0, swait=True, donext=False, dometa2=False)
    _scatter_wait(s0)
    _scatter_wait(s1)

    plsc.subcore_barrier()

    pltpu.sync_copy(out_sh.at[pl.ds(r0, ROWS_A)],
                    out_hbm.at[c, pl.ds(r0, ROWS_A)])

    @pl.when(s == NS - 1)
    def _write_extra():
        pltpu.sync_copy(out_sh.at[pl.ds(NS * ROWS_A, ROWS_EXTRA)],
                        out_hbm.at[c, pl.ds(NS * ROWS_A, ROWS_EXTRA)])


def _sc_edge_kernel(y, gidx, seg, row):
    mesh = plsc.VectorSubcoreMesh(core_axis_name="c", subcore_axis_name="s",
                                  num_cores=NC, num_subcores=NS)
    f = pl.kernel(
        _sc_body,
        out_type=jax.ShapeDtypeStruct((NC, N_NODES, D), jnp.float32),
        mesh=mesh,
        scratch_types=[
            pltpu.VMEM_SHARED((NSEG_PAD,), jnp.float32),
            pltpu.VMEM_SHARED((N_NODES, D), jnp.float32),
            pltpu.VMEM((WSL,), jnp.float32),
            pltpu.VMEM((CH,), jnp.float32),
            pltpu.VMEM((5 * CH,), jnp.int32),
            pltpu.VMEM((5, CH), jnp.int32),
            pltpu.VMEM((CH,), jnp.int32),
            pltpu.VMEM((CH,), jnp.int32),
            pltpu.VMEM((CH,), jnp.int32),
            pltpu.VMEM((CH,), jnp.int32),
            pltpu.VMEM((CH,), jnp.int32),
            pltpu.VMEM((CH,), jnp.int32),
            pltpu.VMEM((CH,), jnp.int32),
            pltpu.VMEM((CH,), jnp.int32),
            pltpu.VMEM((CH,), jnp.int32),
            pltpu.VMEM((CH,), jnp.float32),
            pltpu.VMEM((CH,), jnp.float32),
            pltpu.VMEM((CH,), jnp.float32),
            pltpu.VMEM((CH, D), jnp.float32),
            pltpu.VMEM((CH, D), jnp.float32),
            pltpu.VMEM((CH, D), jnp.float32),
            pltpu.SemaphoreType.DMA,
            pltpu.SemaphoreType.DMA,
            pltpu.SemaphoreType.DMA,
            pltpu.SemaphoreType.DMA,
            pltpu.SemaphoreType.DMA,
            pltpu.SemaphoreType.DMA,
            pltpu.SemaphoreType.DMA,
            pltpu.SemaphoreType.DMA,
            pltpu.SemaphoreType.DMA,
            pltpu.SemaphoreType.DMA,
            pltpu.SemaphoreType.DMA,
            pltpu.SemaphoreType.DMA,
            pltpu.SemaphoreType.DMA,
        ],
        compiler_params=pltpu.CompilerParams(needs_layout_passes=False),
    )
    return f(y, gidx, seg, row)


def kernel(x, edge_index, edge_type, weights):
    row = edge_index[0].astype(jnp.int32)
    col = edge_index[1].astype(jnp.int32)
    et = edge_type.astype(jnp.int32)
    gidx = et * N_NODES + col
    seg = row * T + et
    y = _compute_y(x, weights)
    partials = _sc_edge_kernel(y, gidx, seg, row)
    return _combine(partials)
